# Initial kernel scaffold; baseline (speedup 1.0000x reference)
import jax
import jax.numpy as jnp
from jax.experimental import pallas as pl


def kernel(token_sequence, embedding_sequence, compression_rate):
    raise NotImplementedError("write your pallas kernel here")


# speedup vs baseline: 10.4982x; 10.4982x over previous
import functools

import jax
import jax.numpy as jnp
from jax import lax
from jax.experimental import pallas as pl
from jax.experimental.pallas import tpu as pltpu
from jax.experimental.pallas import tpu_sc as plsc

_L = 16
_ROWS = 64
_COLS = 8192
_CHUNKS = _COLS // _L
_NW = 32
_ROWS_PER_W = _ROWS // _NW
_UNROLL = 8

_IMIN = -(2 ** 31)
_IMAXP = 2 ** 31 - 1


def _chunk_loop(body, carry):
    def outer(i, c):
        for u in range(_UNROLL):
            c = body(i * (_UNROLL * _L) + u * _L, c)
        return c
    return lax.fori_loop(0, _CHUNKS // _UNROLL, outer, carry)


def _tec_body(tok_hbm, kv_hbm, out_hbm, row_v, key_v, mask_v, kv_v):
    wid = lax.axis_index("s") * 2 + lax.axis_index("c")

    pltpu.sync_copy(kv_hbm, kv_v)
    kvec = kv_v[...]
    km1 = kvec - 1

    zeros = jnp.zeros((_L,), jnp.int32)
    ones = zeros + 1
    iota = lax.iota(jnp.int32, _L)
    delta = plsc.cumsum(ones) - iota

    for r in range(_ROWS_PER_W):
        row = wid * _ROWS_PER_W + r
        pltpu.sync_copy(tok_hbm.at[row], row_v)

        def key_body(base, c):
            x = row_v[pl.ds(base, _L)]
            b = lax.bitcast_convert_type(x, jnp.int32) ^ _IMIN
            ks = jnp.where(b < 0, b ^ _IMAXP, b)
            key_v[pl.ds(base, _L)] = ks
            return c
        _chunk_loop(key_body, zeros)
        k0 = key_v[pl.ds(0, _L)]
        key_v[pl.ds(0, _L)] = jnp.where(iota == 0, _IMIN, k0)

        def bit_body(_, st):
            pu, bit = st
            cand_u = pu | bit
            cand = cand_u ^ _IMIN
            def cnt_body(base, cnt):
                m = key_v[pl.ds(base, _L)] < cand
                return cnt + plsc.all_reduce_population_count(m)
            cnt = _chunk_loop(cnt_body, zeros)
            take = cnt <= km1
            return jnp.where(take, cand_u, pu), lax.shift_right_logical(bit, ones)
        pu, _ = lax.fori_loop(0, 32, bit_body, (zeros, zeros + _IMIN))
        t_key = pu ^ _IMIN

        def less_body(base, cnt):
            m = key_v[pl.ds(base, _L)] < t_key
            return cnt + plsc.all_reduce_population_count(m)
        count_less = _chunk_loop(less_body, zeros)
        quota = kvec - count_less

        def mask_body(base, carry):
            c = key_v[pl.ds(base, _L)]
            ltm = c < t_key
            eqm = c == t_key
            eqi = jnp.where(eqm, 1, 0)
            excl = plsc.cumsum(eqi) - eqi * delta + carry
            keep = ltm | (eqm & (excl < quota))
            mask_v[pl.ds(base, _L)] = jnp.where(keep, 1, 0)
            return carry + plsc.all_reduce_population_count(eqm)
        _chunk_loop(mask_body, zeros)

        pltpu.sync_copy(mask_v, out_hbm.at[row])


@jax.jit
def _select_mask(token_sequence, kvec):
    mesh = plsc.VectorSubcoreMesh(core_axis_name="c", subcore_axis_name="s")
    f = pl.kernel(
        _tec_body,
        out_type=jax.ShapeDtypeStruct((_ROWS, _COLS), jnp.int32),
        mesh=mesh,
        scratch_types=[
            pltpu.VMEM((_COLS,), jnp.float32),
            pltpu.VMEM((_COLS,), jnp.int32),
            pltpu.VMEM((_COLS,), jnp.int32),
            pltpu.VMEM((_L,), jnp.int32),
        ],
        compiler_params=pltpu.CompilerParams(needs_layout_passes=False),
    )
    return f(token_sequence, kvec)


def kernel(token_sequence, embedding_sequence, compression_rate):
    seq_len = token_sequence.shape[1]
    c = compression_rate.reshape(-1)[0]
    scaled = seq_len * c
    fs = jnp.floor(scaled)
    k = jnp.where(scaled == fs, seq_len - fs, seq_len - fs - 1.0).astype(jnp.int32)
    k = jnp.maximum(k, 1)
    kvec = jnp.broadcast_to(k, (_L,)).astype(jnp.int32)
    mask = _select_mask(token_sequence, kvec)
    y = mask.astype(bool)
    return (y, y)
